# baseline (speedup 1.0000x reference)
import functools

import jax
import jax.numpy as jnp
from jax import lax
from jax.experimental import pallas as pl
from jax.experimental.pallas import tpu as pltpu
from jax.experimental.pallas import tpu_sc as plsc

_SEQ = 77
_DIM = 4096
_NW = 32
_KPP = 10
_NB = 3
_TNS = 8


def _chunk(i):
    p, kk = divmod(i, _KPP)
    return p, kk * 8, 5 if kk == _KPP - 1 else 8


def _sc_gather(idx, table, nb):
    ppw = nb // _NW
    nch = ppw * _KPP
    mesh = plsc.VectorSubcoreMesh(core_axis_name="c", subcore_axis_name="s")

    @functools.partial(
        pl.kernel,
        out_type=jax.ShapeDtypeStruct((nb, _SEQ, _DIM), jnp.float32),
        mesh=mesh,
        scratch_types=[
            pltpu.VMEM((16,), jnp.int32),
            pltpu.VMEM((_NB, 8, _DIM), jnp.float32),
            pltpu.SemaphoreType.DMA((_NB,)),
            pltpu.SemaphoreType.DMA((_NB,)),
        ],
    )
    def k(idx_hbm, table_hbm, out_hbm, pids, buf, gsem, wsem):
        w = lax.axis_index("s") * 2 + lax.axis_index("c")
        base_p = w * ppw

        blk = (base_p // 8) * 8
        off = base_p - blk
        pltpu.sync_copy(idx_hbm.at[pl.ds(blk, 8)], pids.at[pl.ds(0, 8)])
        raw = pids[...]
        lanes = jax.lax.rem(lax.iota(jnp.int32, 16) + off, 16)
        pv16 = raw.at[lanes].get(mode="promise_in_bounds")

        def start_g(i):
            p, r0, nr = _chunk(i)
            pltpu.async_copy(
                table_hbm.at[pv16[p], pl.ds(r0, nr)],
                buf.at[i % _NB, pl.ds(0, nr)], gsem.at[i % _NB])

        def wait_g(i):
            _, r0, nr = _chunk(i)
            pltpu.make_async_copy(
                table_hbm.at[0, pl.ds(r0, nr)],
                buf.at[i % _NB, pl.ds(0, nr)], gsem.at[i % _NB]).wait()

        def start_w(i):
            p, r0, nr = _chunk(i)
            pltpu.async_copy(
                buf.at[i % _NB, pl.ds(0, nr)],
                out_hbm.at[base_p + p, pl.ds(r0, nr)], wsem.at[i % _NB])

        def wait_w(i):
            p, r0, nr = _chunk(i)
            pltpu.make_async_copy(
                buf.at[i % _NB, pl.ds(0, nr)],
                out_hbm.at[0, pl.ds(r0, nr)], wsem.at[i % _NB]).wait()

        for i in range(_NB):
            start_g(i)
        for i in range(nch):
            wait_g(i)
            start_w(i)
            if i + _NB < nch:
                wait_w(i)
                start_g(i + _NB)
        for i in range(nch - _NB, nch):
            wait_w(i)

    return k(idx, table)


def _tc_gather(idx, emb, nb):
    nt = nb // _TNS

    def body(idx_ref, in_ref, out_ref, bufs, gsem, wsem):
        def start_g(i, s):
            pltpu.make_async_copy(
                in_ref.at[idx_ref[i]], bufs.at[s], gsem.at[s]).start()

        def wait_g(s):
            pltpu.make_async_copy(
                in_ref.at[0], bufs.at[s], gsem.at[s]).wait()

        def start_w(i, s):
            pltpu.make_async_copy(
                bufs.at[s], out_ref.at[i], wsem.at[s]).start()

        def wait_w(s):
            pltpu.make_async_copy(
                bufs.at[0], out_ref.at[0], wsem.at[s]).wait()

        for s in range(_TNS):
            start_g(s, s)

        def loop(t, carry):
            for s in range(_TNS):
                wait_g(s)
                start_w(t * _TNS + s, s)
            for s in range(_TNS):
                wait_w(s)

                @pl.when(t + 1 < nt)
                def _():
                    start_g((t + 1) * _TNS + s, s)

            return carry

        jax.lax.fori_loop(0, nt, loop, 0)

    grid_spec = pltpu.PrefetchScalarGridSpec(
        num_scalar_prefetch=1,
        grid=(),
        in_specs=[pl.BlockSpec(memory_space=pltpu.MemorySpace.HBM)],
        out_specs=pl.BlockSpec(memory_space=pltpu.MemorySpace.HBM),
        scratch_shapes=[
            pltpu.VMEM((_TNS, _SEQ, _DIM), jnp.float32),
            pltpu.SemaphoreType.DMA((_TNS,)),
            pltpu.SemaphoreType.DMA((_TNS,)),
        ],
    )
    return pl.pallas_call(
        body,
        grid_spec=grid_spec,
        out_shape=jax.ShapeDtypeStruct((nb, _SEQ, _DIM), jnp.float32),
    )(idx, emb)


_SC_FRAC = 128


def kernel(prompt_idx, embeddings):
    idx = prompt_idx.astype(jnp.int32)
    sc_out = _sc_gather(idx[:_SC_FRAC], embeddings, _SC_FRAC)
    tc_out = _tc_gather(idx[_SC_FRAC:], embeddings, 256 - _SC_FRAC)
    return jnp.concatenate([sc_out, tc_out], axis=0)


# speedup vs baseline: 1.1023x; 1.1023x over previous
import functools

import jax
import jax.numpy as jnp
from jax import lax
from jax.experimental import pallas as pl
from jax.experimental.pallas import tpu as pltpu
from jax.experimental.pallas import tpu_sc as plsc

_NUM_PROMPTS = 1000
_SEQ_LEN = 77
_TEXT_DIM = 4096
_BATCH = 256

_NW = 32
_PPW = _BATCH // _NW
_KPP = 10
_NCH = _PPW * _KPP
_NB = 3


def _chunk(i):
    p, kk = divmod(i, _KPP)
    return p, kk * 8, 5 if kk == _KPP - 1 else 8


def _sc_gather(idx, table):
    mesh = plsc.VectorSubcoreMesh(core_axis_name="c", subcore_axis_name="s")

    @functools.partial(
        pl.kernel,
        out_type=jax.ShapeDtypeStruct((_BATCH, _SEQ_LEN, _TEXT_DIM),
                                      jnp.float32),
        mesh=mesh,
        scratch_types=[
            pltpu.VMEM((16,), jnp.int32),
            pltpu.VMEM((_NB, 8, _TEXT_DIM), jnp.float32),
            pltpu.SemaphoreType.DMA((_NB,)),
            pltpu.SemaphoreType.DMA((_NB,)),
        ],
    )
    def k(idx_hbm, table_hbm, out_hbm, pids, buf, gsem, wsem):
        w = lax.axis_index("s") * 2 + lax.axis_index("c")
        base_p = w * _PPW

        pltpu.sync_copy(idx_hbm.at[pl.ds(base_p, _PPW)],
                        pids.at[pl.ds(0, _PPW)])
        pv16 = pids[...]

        def start_g(i):
            p, r0, nr = _chunk(i)
            pltpu.async_copy(
                table_hbm.at[pv16[p], pl.ds(r0, nr)],
                buf.at[i % _NB, pl.ds(0, nr)], gsem.at[i % _NB])

        def wait_g(i):
            _, r0, nr = _chunk(i)
            pltpu.make_async_copy(
                table_hbm.at[0, pl.ds(r0, nr)],
                buf.at[i % _NB, pl.ds(0, nr)], gsem.at[i % _NB]).wait()

        def start_w(i):
            p, r0, nr = _chunk(i)
            pltpu.async_copy(
                buf.at[i % _NB, pl.ds(0, nr)],
                out_hbm.at[base_p + p, pl.ds(r0, nr)], wsem.at[i % _NB])

        def wait_w(i):
            p, r0, nr = _chunk(i)
            pltpu.make_async_copy(
                buf.at[i % _NB, pl.ds(0, nr)],
                out_hbm.at[0, pl.ds(r0, nr)], wsem.at[i % _NB]).wait()

        for i in range(_NB):
            start_g(i)
        for i in range(_NCH):
            wait_g(i)
            start_w(i)
            if i + _NB < _NCH:
                wait_w(i)
                start_g(i + _NB)
        for i in range(_NCH - _NB, _NCH):
            wait_w(i)

    return k(idx, table)


def kernel(prompt_idx, embeddings):
    idx = prompt_idx.astype(jnp.int32)
    return _sc_gather(idx, embeddings)
